# baseline (speedup 1.0000x reference)
import functools

import jax
import jax.numpy as jnp
from jax import lax
from jax.experimental import pallas as pl
from jax.experimental.pallas import tpu as pltpu
from jax.experimental.pallas import tpu_sc as plsc

N = 160000
D = 128
NSEG = 10000


BLK = 8192
ROWS = BLK // 128
NROW = 1280


LOG2E = 1.4426950408889634
LN2 = 0.6931471805599453


def _ssp(x):
    return jnp.maximum(x, 0.0) + jnp.log(1.0 + jnp.exp(-jnp.abs(x))) - LN2


def _dot16(a, b):
    return jnp.dot(a, b, preferred_element_type=jnp.float32)


def _mlp_body(s_ref, v_ref, wmix1t_ref, w11t_ref, b11_ref, w21t_ref, b21_ref,
              wm2_ref, a2t_ref, w1l_ref, b12_ref, w22_ref, b22_ref, out_ref):
    s = s_ref[...]
    wmix1t = wmix1t_ref[...]
    vsq = None
    wv = []
    for c in range(3):
        vc = v_ref[c]
        vm = _dot16(vc, wmix1t)
        V = vm[:, :D]
        wv.append(vm[:, D:])
        vsq = V * V if vsq is None else vsq + V * V
    vn = jnp.sqrt(vsq)
    ctx = jnp.concatenate([s, vn], axis=1)
    x = _ssp(_dot16(ctx, w11t_ref[...]) + b11_ref[...])
    x2 = _dot16(x, w21t_ref[...]) + b21_ref[...]
    s1 = _ssp(x2[:, :D])
    xv = x2[:, D:]
    m = xv * wm2_ref[...]
    v2sq = None
    for c in range(3):
        v2c = jnp.sum(m * wv[c], axis=1, keepdims=True)
        v2sq = v2c * v2c if v2sq is None else v2sq + v2c * v2c
    vn2 = jnp.sqrt(v2sq)
    x3 = _ssp(_dot16(s1, a2t_ref[...]) + vn2 * w1l_ref[...] + b12_ref[...])
    yi = lax.dot_general(x3.reshape(ROWS, 128, D), w22_ref[...].reshape(D),
                         (((2,), (0,)), ((), ())),
                         preferred_element_type=jnp.float32) + b22_ref[0, 0]
    out_ref[...] = yi


def _atom_mlp(s, v3, wmix1t, w11t, b11, w21t, b21, wm2, a2t, w1l, b12, w22, b22):
    nblk = NROW // ROWS
    full = lambda shape: pl.BlockSpec(shape, lambda i: (0, 0))
    return pl.pallas_call(
        _mlp_body,
        grid=(nblk,),
        in_specs=[
            pl.BlockSpec((BLK, D), lambda i: (i, 0)),
            pl.BlockSpec((3, BLK, D), lambda i: (0, i, 0)),
            full((D, 2 * D)),
            full((2 * D, D)),
            full((1, D)),
            full((D, 2 * D)),
            full((1, 2 * D)),
            full((1, D)),
            full((D, D)),
            full((1, D)),
            full((1, D)),
            full((1, D)),
            full((1, 1)),
        ],
        out_specs=pl.BlockSpec((ROWS, 128), lambda i: (i, 0)),
        out_shape=jax.ShapeDtypeStruct((NROW, 128), jnp.float32),
        compiler_params=pltpu.CompilerParams(
            dimension_semantics=("parallel",),
            vmem_limit_bytes=100 * 1024 * 1024),
    )(s, v3, wmix1t, w11t, b11, w21t, b21, wm2, a2t, w1l, b12, w22, b22)


NW = 16
CHUNK = N // NW
SEG_PAD = 10240
OWN = SEG_PAD // NW


def _segsum_body(yi_hbm, b_hbm, out_hbm, yiv, bv, acc, buf, res, slab):
    cid = lax.axis_index("c")
    sid = lax.axis_index("s")
    iota = lax.iota(jnp.int32, 16)
    take = lambda arr, idx: arr.at[idx].get(mode="promise_in_bounds")

    @pl.when(cid == 0)
    def _phase1():
        base = sid * CHUNK
        pltpu.sync_copy(yi_hbm.at[pl.ds(base, CHUNK)], yiv)
        pltpu.sync_copy(b_hbm.at[pl.ds(base, CHUNK)], bv)
        zf = jnp.zeros((16,), jnp.float32)

        def zbody(i, _):
            acc[pl.ds(pl.multiple_of(i * 16, 16), 16)] = zf
            return 0
        lax.fori_loop(0, SEG_PAD // 16, zbody, 0, unroll=4)

        def body(i, _):
            off = pl.multiple_of(i * 16, 16)
            y = yiv[pl.ds(off, 16)]
            b = bv[pl.ds(off, 16)]
            c = plsc.cumsum(y)
            bp = take(b, jnp.maximum(iota - 1, 0))
            first = (b != bp) | (iota == 0)
            bn = take(b, jnp.minimum(iota + 1, 15))
            last = (b != bn) | (iota == 15)
            g = plsc.cummax(jnp.where(first, iota, -1))
            excl = take(c - y, g)
            plsc.addupdate_scatter(acc, [b], c - excl, mask=last)
            return 0
        lax.fori_loop(0, CHUNK // 16, body, 0, unroll=4)
        for t in range(NW):
            pltpu.sync_copy(acc.at[pl.ds(t * OWN, OWN)], slab.at[t, sid])

    plsc.subcore_barrier()

    @pl.when(cid == 0)
    def _phase2():
        sid2 = lax.axis_index("s")
        pltpu.sync_copy(slab.at[sid2], buf)

        def mbody(j, _):
            off = pl.multiple_of(j * 16, 16)
            v = buf[0, pl.ds(off, 16)]
            for w in range(1, NW):
                v = v + buf[w, pl.ds(off, 16)]
            res[pl.ds(off, 16)] = v
            return 0
        lax.fori_loop(0, OWN // 16, mbody, 0)
        pltpu.sync_copy(res, out_hbm.at[pl.ds(sid2 * OWN, OWN)])


def _segment_sum(yi, batch):
    mesh = plsc.VectorSubcoreMesh(core_axis_name="c", subcore_axis_name="s")
    return pl.kernel(
        _segsum_body,
        out_type=jax.ShapeDtypeStruct((SEG_PAD,), jnp.float32),
        mesh=mesh,
        compiler_params=pltpu.CompilerParams(needs_layout_passes=False),
        scratch_types=[
            pltpu.VMEM((CHUNK,), jnp.float32),
            pltpu.VMEM((CHUNK,), jnp.int32),
            pltpu.VMEM((SEG_PAD,), jnp.float32),
            pltpu.VMEM((NW, OWN), jnp.float32),
            pltpu.VMEM((OWN,), jnp.float32),
            pltpu.VMEM_SHARED((NW, NW, OWN), jnp.float32),
        ],
    )(yi, batch)


def kernel(representation, vector_representation, z, batch, Wmix1, W1_1, b1_1,
           W2_1, b2_1, Wmix2, W1_2, b1_2, W2_2, b2_2):
    del z
    v3 = jnp.transpose(vector_representation, (1, 0, 2))
    yi2 = _atom_mlp(
        representation, v3,
        Wmix1.T,
        W1_1.T,
        b1_1.reshape(1, D),
        W2_1.T,
        b2_1.reshape(1, 2 * D),
        Wmix2[0].reshape(1, D),
        W1_2[:, :D].T,
        W1_2[:, D].reshape(1, D),
        b1_2.reshape(1, D),
        W2_2[0].reshape(1, D),
        b2_2[0].reshape(1, 1),
    )
    y = _segment_sum(yi2.reshape(NROW * 128), batch.astype(jnp.int32))
    return y[:NSEG].reshape(NSEG, 1)


# speedup vs baseline: 1.0144x; 1.0144x over previous
import functools

import jax
import jax.numpy as jnp
from jax import lax
from jax.experimental import pallas as pl
from jax.experimental.pallas import tpu as pltpu
from jax.experimental.pallas import tpu_sc as plsc

N = 160000
D = 128
NSEG = 10000


BLK = 8192
ROWS = BLK // 128
NROW = 1280


LOG2E = 1.4426950408889634
LN2 = 0.6931471805599453


def _ssp(x):
    return jnp.maximum(x, 0.0) + jnp.log(1.0 + jnp.exp(-jnp.abs(x))) - LN2


def _dot16(a, b):
    return jnp.dot(a, b, preferred_element_type=jnp.float32)


def _mlp_body(s_ref, v_ref, wmix1t_ref, w11t_ref, b11_ref, w21t_ref, b21_ref,
              wm2_ref, a2t_ref, w1l_ref, b12_ref, w22_ref, b22_ref, out_ref):
    s = s_ref[...]
    wmix1t = wmix1t_ref[...]
    vsq = None
    wv = []
    for c in range(3):
        vc = v_ref[c]
        V = _dot16(vc, wmix1t[:, :D])
        wv.append(_dot16(vc, wmix1t[:, D:]))
        vsq = V * V if vsq is None else vsq + V * V
    vn = jnp.sqrt(vsq)
    ctx = jnp.concatenate([s, vn], axis=1)
    x = _ssp(_dot16(ctx, w11t_ref[...]) + b11_ref[...])
    x2 = _dot16(x, w21t_ref[...]) + b21_ref[...]
    s1 = _ssp(x2[:, :D])
    xv = x2[:, D:]
    m = xv * wm2_ref[...]
    v2sq = None
    for c in range(3):
        v2c = jnp.sum(m * wv[c], axis=1, keepdims=True)
        v2sq = v2c * v2c if v2sq is None else v2sq + v2c * v2c
    vn2 = jnp.sqrt(v2sq)
    x3 = _ssp(_dot16(s1, a2t_ref[...]) + vn2 * w1l_ref[...] + b12_ref[...])
    yi = lax.dot_general(x3.reshape(ROWS, 128, D), w22_ref[...].reshape(D),
                         (((2,), (0,)), ((), ())),
                         preferred_element_type=jnp.float32) + b22_ref[0, 0]
    out_ref[...] = yi


def _atom_mlp(s, v3, wmix1t, w11t, b11, w21t, b21, wm2, a2t, w1l, b12, w22, b22):
    nblk = NROW // ROWS
    full = lambda shape: pl.BlockSpec(shape, lambda i: (0, 0))
    return pl.pallas_call(
        _mlp_body,
        grid=(nblk,),
        in_specs=[
            pl.BlockSpec((BLK, D), lambda i: (i, 0)),
            pl.BlockSpec((3, BLK, D), lambda i: (0, i, 0)),
            full((D, 2 * D)),
            full((2 * D, D)),
            full((1, D)),
            full((D, 2 * D)),
            full((1, 2 * D)),
            full((1, D)),
            full((D, D)),
            full((1, D)),
            full((1, D)),
            full((1, D)),
            full((1, 1)),
        ],
        out_specs=pl.BlockSpec((ROWS, 128), lambda i: (i, 0)),
        out_shape=jax.ShapeDtypeStruct((NROW, 128), jnp.float32),
        compiler_params=pltpu.CompilerParams(
            dimension_semantics=("parallel",),
            vmem_limit_bytes=100 * 1024 * 1024),
    )(s, v3, wmix1t, w11t, b11, w21t, b21, wm2, a2t, w1l, b12, w22, b22)


NW = 16
CHUNK = N // NW
SEG_PAD = 10240
OWN = SEG_PAD // NW


def _segsum_body(yi_hbm, b_hbm, out_hbm, yiv, bv, acc, buf, res, slab):
    cid = lax.axis_index("c")
    sid = lax.axis_index("s")
    iota = lax.iota(jnp.int32, 16)
    take = lambda arr, idx: arr.at[idx].get(mode="promise_in_bounds")

    @pl.when(cid == 0)
    def _phase1():
        base = sid * CHUNK
        pltpu.sync_copy(yi_hbm.at[pl.ds(base, CHUNK)], yiv)
        pltpu.sync_copy(b_hbm.at[pl.ds(base, CHUNK)], bv)
        zf = jnp.zeros((16,), jnp.float32)

        def zbody(i, _):
            acc[pl.ds(pl.multiple_of(i * 16, 16), 16)] = zf
            return 0
        lax.fori_loop(0, SEG_PAD // 16, zbody, 0, unroll=4)

        def body(i, _):
            off = pl.multiple_of(i * 16, 16)
            y = yiv[pl.ds(off, 16)]
            b = bv[pl.ds(off, 16)]
            c = plsc.cumsum(y)
            bp = take(b, jnp.maximum(iota - 1, 0))
            first = (b != bp) | (iota == 0)
            bn = take(b, jnp.minimum(iota + 1, 15))
            last = (b != bn) | (iota == 15)
            g = plsc.cummax(jnp.where(first, iota, -1))
            excl = take(c - y, g)
            plsc.addupdate_scatter(acc, [b], c - excl, mask=last)
            return 0
        lax.fori_loop(0, CHUNK // 16, body, 0, unroll=4)
        for t in range(NW):
            pltpu.sync_copy(acc.at[pl.ds(t * OWN, OWN)], slab.at[t, sid])

    plsc.subcore_barrier()

    @pl.when(cid == 0)
    def _phase2():
        sid2 = lax.axis_index("s")
        pltpu.sync_copy(slab.at[sid2], buf)

        def mbody(j, _):
            off = pl.multiple_of(j * 16, 16)
            v = buf[0, pl.ds(off, 16)]
            for w in range(1, NW):
                v = v + buf[w, pl.ds(off, 16)]
            res[pl.ds(off, 16)] = v
            return 0
        lax.fori_loop(0, OWN // 16, mbody, 0)
        pltpu.sync_copy(res, out_hbm.at[pl.ds(sid2 * OWN, OWN)])


def _segment_sum(yi, batch):
    mesh = plsc.VectorSubcoreMesh(core_axis_name="c", subcore_axis_name="s")
    return pl.kernel(
        _segsum_body,
        out_type=jax.ShapeDtypeStruct((SEG_PAD,), jnp.float32),
        mesh=mesh,
        compiler_params=pltpu.CompilerParams(needs_layout_passes=False),
        scratch_types=[
            pltpu.VMEM((CHUNK,), jnp.float32),
            pltpu.VMEM((CHUNK,), jnp.int32),
            pltpu.VMEM((SEG_PAD,), jnp.float32),
            pltpu.VMEM((NW, OWN), jnp.float32),
            pltpu.VMEM((OWN,), jnp.float32),
            pltpu.VMEM_SHARED((NW, NW, OWN), jnp.float32),
        ],
    )(yi, batch)


def kernel(representation, vector_representation, z, batch, Wmix1, W1_1, b1_1,
           W2_1, b2_1, Wmix2, W1_2, b1_2, W2_2, b2_2):
    del z
    v3 = jnp.transpose(vector_representation, (1, 0, 2))
    yi2 = _atom_mlp(
        representation, v3,
        Wmix1.T,
        W1_1.T,
        b1_1.reshape(1, D),
        W2_1.T,
        b2_1.reshape(1, 2 * D),
        Wmix2[0].reshape(1, D),
        W1_2[:, :D].T,
        W1_2[:, D].reshape(1, D),
        b1_2.reshape(1, D),
        W2_2[0].reshape(1, D),
        b2_2[0].reshape(1, 1),
    )
    y = _segment_sum(yi2.reshape(NROW * 128), batch.astype(jnp.int32))
    return y[:NSEG].reshape(NSEG, 1)
